# Initial kernel scaffold; baseline (speedup 1.0000x reference)
import jax
import jax.numpy as jnp
from jax.experimental import pallas as pl


def kernel(x, stamp, time_matrix, tok_table, weekday_table, day_table, month_table, K_table, V_table, gamma, beta):
    raise NotImplementedError("write your pallas kernel here")


# speedup vs baseline: 3.5541x; 3.5541x over previous
import functools

import jax
import jax.numpy as jnp
from jax import lax
from jax.experimental import pallas as pl
from jax.experimental.pallas import tpu as pltpu
from jax.experimental.pallas import tpu_sc as plsc

_EPS = 1e-12
_CH = 128


def _sc_gather(x_flat, midx, didx, tm_flat, tok_table, maug, daug, K_table, V_table):
    n_tok = x_flat.shape[0]
    n_kv = tm_flat.shape[0]
    d = tok_table.shape[1]
    info = plsc.get_sparse_core_info()
    nc, ns = info.num_cores, info.num_subcores
    nw = nc * ns
    tok_pw = n_tok // nw
    kv_pw = n_kv // nw
    n_tok_ch = tok_pw // _CH
    n_kv_ch = kv_pw // _CH
    mesh = plsc.VectorSubcoreMesh(core_axis_name="c", subcore_axis_name="s")

    @functools.partial(
        pl.kernel,
        out_type=(
            jax.ShapeDtypeStruct((n_tok, d), jnp.float32),
            jax.ShapeDtypeStruct((n_kv, d), jnp.float32),
            jax.ShapeDtypeStruct((n_kv, d), jnp.float32),
        ),
        mesh=mesh,
        compiler_params=pltpu.CompilerParams(use_tc_tiling_on_sc=False),
        scratch_types=[
            pltpu.VMEM((tok_pw,), jnp.int32),
            pltpu.VMEM((tok_pw,), jnp.int32),
            pltpu.VMEM((tok_pw,), jnp.int32),
            pltpu.VMEM((kv_pw,), jnp.int32),
            pltpu.VMEM((_CH, d), jnp.float32),
            pltpu.VMEM((_CH, d), jnp.float32),
            pltpu.VMEM((_CH, d), jnp.float32),
            pltpu.SemaphoreType.DMA,
        ],
    )
    def k(x_hbm, m_hbm, d_hbm, tm_hbm, tok_hbm, maug_hbm, daug_hbm, kt_hbm, vt_hbm,
          h_out, k_out, v_out, xi_v, mi_v, di_v, tmi_v, hbuf, kbuf, vbuf, sem):
        wid = lax.axis_index("s") * nc + lax.axis_index("c")
        tbase = wid * tok_pw
        kvbase = wid * kv_pw
        pltpu.sync_copy(x_hbm.at[pl.ds(tbase, tok_pw)], xi_v)
        pltpu.sync_copy(m_hbm.at[pl.ds(tbase, tok_pw)], mi_v)
        pltpu.sync_copy(d_hbm.at[pl.ds(tbase, tok_pw)], di_v)
        pltpu.sync_copy(tm_hbm.at[pl.ds(kvbase, kv_pw)], tmi_v)

        def tok_body(c, carry):
            off = c * _CH
            pltpu.async_copy(tok_hbm.at[xi_v.at[pl.ds(off, _CH)]], hbuf, sem).wait()
            pltpu.async_copy(maug_hbm.at[mi_v.at[pl.ds(off, _CH)]], hbuf, sem, add=True).wait()
            pltpu.async_copy(daug_hbm.at[di_v.at[pl.ds(off, _CH)]], hbuf, sem, add=True).wait()
            pltpu.sync_copy(hbuf, h_out.at[pl.ds(tbase + off, _CH)])
            return carry

        lax.fori_loop(0, n_tok_ch, tok_body, 0)

        def kv_body(c, carry):
            off = c * _CH
            idx = tmi_v.at[pl.ds(off, _CH)]
            pltpu.async_copy(kt_hbm.at[idx], kbuf, sem).wait()
            pltpu.async_copy(vt_hbm.at[idx], vbuf, sem).wait()
            pltpu.sync_copy(kbuf, k_out.at[pl.ds(kvbase + off, _CH)])
            pltpu.sync_copy(vbuf, v_out.at[pl.ds(kvbase + off, _CH)])
            return carry

        lax.fori_loop(0, n_kv_ch, kv_body, 0)

    return k(x_flat, midx, didx, tm_flat, tok_table, maug, daug, K_table, V_table)


def _layernorm_tc(h, gamma, beta):
    n, d = h.shape
    blk = 1024

    def body(h_ref, g_ref, b_ref, o_ref):
        hv = h_ref[...]
        u = jnp.mean(hv, axis=-1, keepdims=True)
        c = hv - u
        s = jnp.mean(c * c, axis=-1, keepdims=True)
        o_ref[...] = g_ref[...] * (c * lax.rsqrt(s + _EPS)) + b_ref[...]

    return pl.pallas_call(
        body,
        grid=(n // blk,),
        in_specs=[
            pl.BlockSpec((blk, d), lambda i: (i, 0)),
            pl.BlockSpec((1, d), lambda i: (0, 0)),
            pl.BlockSpec((1, d), lambda i: (0, 0)),
        ],
        out_specs=pl.BlockSpec((blk, d), lambda i: (i, 0)),
        out_shape=jax.ShapeDtypeStruct((n, d), jnp.float32),
    )(h, gamma.reshape(1, d), beta.reshape(1, d))


def kernel(x, stamp, time_matrix, tok_table, weekday_table, day_table, month_table, K_table, V_table, gamma, beta):
    b, l = x.shape
    d = tok_table.shape[1]

    x_flat = x.reshape(-1)
    tm_flat = time_matrix.reshape(-1)
    m_sent = month_table.shape[0]
    d_sent = day_table.shape[0]
    midx = jnp.concatenate(
        [jnp.full((b, 1), m_sent, jnp.int32), stamp[:, :, 0]], axis=1).reshape(-1)
    didx = jnp.concatenate(
        [jnp.full((b, 1), d_sent, jnp.int32), stamp[:, :, 1]], axis=1).reshape(-1)
    maug = jnp.concatenate(
        [month_table, jnp.zeros((3, d), month_table.dtype)], axis=0)
    daug = jnp.concatenate(
        [day_table, jnp.zeros((8, d), day_table.dtype)], axis=0)

    h, kout, vout = _sc_gather(x_flat, midx, didx, tm_flat, tok_table,
                               maug, daug, K_table, V_table)
    out = _layernorm_tc(h, gamma, beta)
    return (
        out.reshape(b, l, d),
        kout.reshape(b, l, l, d),
        vout.reshape(b, l, l, d),
    )


# speedup vs baseline: 3.7318x; 1.0500x over previous
import functools

import jax
import jax.numpy as jnp
from jax import lax
from jax.experimental import pallas as pl
from jax.experimental.pallas import tpu as pltpu
from jax.experimental.pallas import tpu_sc as plsc

_EPS = 1e-12
_CH = 128
_NBUF = 4


def _sc_gather(x_flat, midx, didx, tm_flat, tok_table, maug, daug, K_table, V_table):
    n_tok = x_flat.shape[0]
    n_kv = tm_flat.shape[0]
    d = tok_table.shape[1]
    info = plsc.get_sparse_core_info()
    nc, ns = info.num_cores, info.num_subcores
    nw = nc * ns
    tok_pw = n_tok // nw
    kv_pw = n_kv // nw
    n_tok_ch = tok_pw // _CH
    n_kv_ch = kv_pw // _CH
    mesh = plsc.VectorSubcoreMesh(core_axis_name="c", subcore_axis_name="s")

    @functools.partial(
        pl.kernel,
        out_type=(
            jax.ShapeDtypeStruct((n_tok, d), jnp.float32),
            jax.ShapeDtypeStruct((n_kv, d), jnp.float32),
            jax.ShapeDtypeStruct((n_kv, d), jnp.float32),
        ),
        mesh=mesh,
        compiler_params=pltpu.CompilerParams(use_tc_tiling_on_sc=False),
        scratch_types=[
            pltpu.VMEM((tok_pw,), jnp.int32),
            pltpu.VMEM((tok_pw,), jnp.int32),
            pltpu.VMEM((tok_pw,), jnp.int32),
            pltpu.VMEM((kv_pw,), jnp.int32),
            [pltpu.VMEM((_CH, d), jnp.float32)] * _NBUF,
            [pltpu.VMEM((_CH, d), jnp.float32)] * _NBUF,
            [pltpu.SemaphoreType.DMA] * _NBUF,
            [pltpu.SemaphoreType.DMA] * _NBUF,
        ],
    )
    def k(x_hbm, m_hbm, d_hbm, tm_hbm, tok_hbm, maug_hbm, daug_hbm, kt_hbm, vt_hbm,
          h_out, k_out, v_out, xi_v, mi_v, di_v, tmi_v, kbufs, vbufs, gsems, wsems):
        wid = lax.axis_index("s") * nc + lax.axis_index("c")
        tbase = wid * tok_pw
        kvbase = wid * kv_pw
        pltpu.sync_copy(x_hbm.at[pl.ds(tbase, tok_pw)], xi_v)
        pltpu.sync_copy(m_hbm.at[pl.ds(tbase, tok_pw)], mi_v)
        pltpu.sync_copy(d_hbm.at[pl.ds(tbase, tok_pw)], di_v)
        pltpu.sync_copy(tm_hbm.at[pl.ds(kvbase, kv_pw)], tmi_v)

        def wait_write(buf, out_hbm, sem):
            pltpu.make_async_copy(buf, out_hbm.at[pl.ds(0, _CH)], sem).wait()

        for c in range(n_tok_ch):
            i = c % 2
            off = c * _CH
            if c >= 2:
                wait_write(kbufs[i], h_out, wsems[i])
            pltpu.async_copy(tok_hbm.at[xi_v.at[pl.ds(off, _CH)]], kbufs[i], gsems[i]).wait()
            a = pltpu.async_copy(maug_hbm.at[mi_v.at[pl.ds(off, _CH)]], kbufs[i], gsems[i], add=True)
            b = pltpu.async_copy(daug_hbm.at[di_v.at[pl.ds(off, _CH)]], kbufs[i], gsems[i], add=True)
            a.wait()
            b.wait()
            pltpu.async_copy(kbufs[i], h_out.at[pl.ds(tbase + off, _CH)], wsems[i])
        for c in range(max(0, n_tok_ch - 2), n_tok_ch):
            wait_write(kbufs[c % 2], h_out, wsems[c % 2])

        def kv_body(c, carry):
            base = c * _NBUF * _CH
            gathers = []
            for i in range(_NBUF):
                off = base + i * _CH
                idx = tmi_v.at[pl.ds(off, _CH)]

                @pl.when(c > 0)
                def _():
                    wait_write(kbufs[i], k_out, wsems[i])
                    wait_write(vbufs[i], v_out, wsems[i])

                gathers.append(pltpu.async_copy(kt_hbm.at[idx], kbufs[i], gsems[i]))
                gathers.append(pltpu.async_copy(vt_hbm.at[idx], vbufs[i], gsems[i]))
            for i in range(_NBUF):
                off = base + i * _CH
                gathers[2 * i].wait()
                gathers[2 * i + 1].wait()
                pltpu.async_copy(kbufs[i], k_out.at[pl.ds(kvbase + off, _CH)], wsems[i])
                pltpu.async_copy(vbufs[i], v_out.at[pl.ds(kvbase + off, _CH)], wsems[i])
            return carry

        lax.fori_loop(0, n_kv_ch // _NBUF, kv_body, 0)
        for i in range(_NBUF):
            wait_write(kbufs[i], k_out, wsems[i])
            wait_write(vbufs[i], v_out, wsems[i])

    return k(x_flat, midx, didx, tm_flat, tok_table, maug, daug, K_table, V_table)


def _layernorm_tc(h, gamma, beta):
    n, d = h.shape
    blk = 1024

    def body(h_ref, g_ref, b_ref, o_ref):
        hv = h_ref[...]
        u = jnp.mean(hv, axis=-1, keepdims=True)
        c = hv - u
        s = jnp.mean(c * c, axis=-1, keepdims=True)
        o_ref[...] = g_ref[...] * (c * lax.rsqrt(s + _EPS)) + b_ref[...]

    return pl.pallas_call(
        body,
        grid=(n // blk,),
        in_specs=[
            pl.BlockSpec((blk, d), lambda i: (i, 0)),
            pl.BlockSpec((1, d), lambda i: (0, 0)),
            pl.BlockSpec((1, d), lambda i: (0, 0)),
        ],
        out_specs=pl.BlockSpec((blk, d), lambda i: (i, 0)),
        out_shape=jax.ShapeDtypeStruct((n, d), jnp.float32),
    )(h, gamma.reshape(1, d), beta.reshape(1, d))


def kernel(x, stamp, time_matrix, tok_table, weekday_table, day_table, month_table, K_table, V_table, gamma, beta):
    b, l = x.shape
    d = tok_table.shape[1]

    x_flat = x.reshape(-1)
    tm_flat = time_matrix.reshape(-1)
    m_sent = month_table.shape[0]
    d_sent = day_table.shape[0]
    midx = jnp.concatenate(
        [jnp.full((b, 1), m_sent, jnp.int32), stamp[:, :, 0]], axis=1).reshape(-1)
    didx = jnp.concatenate(
        [jnp.full((b, 1), d_sent, jnp.int32), stamp[:, :, 1]], axis=1).reshape(-1)
    maug = jnp.concatenate(
        [month_table, jnp.zeros((3, d), month_table.dtype)], axis=0)
    daug = jnp.concatenate(
        [day_table, jnp.zeros((8, d), day_table.dtype)], axis=0)

    h, kout, vout = _sc_gather(x_flat, midx, didx, tm_flat, tok_table,
                               maug, daug, K_table, V_table)
    out = _layernorm_tc(h, gamma, beta)
    return (
        out.reshape(b, l, d),
        kout.reshape(b, l, l, d),
        vout.reshape(b, l, l, d),
    )


# speedup vs baseline: 4.5096x; 1.2084x over previous
import functools

import jax
import jax.numpy as jnp
from jax import lax
from jax.experimental import pallas as pl
from jax.experimental.pallas import tpu as pltpu
from jax.experimental.pallas import tpu_sc as plsc

_EPS = 1e-12
_CH = 128


def _sc_gather(x_flat, midx, didx, tm_lb, tok_table, maug, daug, kvt):
    n_tok = x_flat.shape[0]
    d = tok_table.shape[1]
    n_slab, b = tm_lb.shape
    vrows = kvt.shape[1]
    kvt_flat = kvt.reshape(-1)
    info = plsc.get_sparse_core_info()
    nc, ns = info.num_cores, info.num_subcores
    nw = nc * ns
    tok_pw = n_tok // nw
    n_tok_ch = tok_pw // _CH
    n_units = 2 * n_slab // nw
    hd = d // 2
    nbg = b // 16
    mesh = plsc.VectorSubcoreMesh(core_axis_name="c", subcore_axis_name="s")

    @functools.partial(
        pl.kernel,
        out_type=(
            jax.ShapeDtypeStruct((n_tok, d), jnp.float32),
            jax.ShapeDtypeStruct((n_slab, d, b), jnp.float32),
            jax.ShapeDtypeStruct((n_slab, d, b), jnp.float32),
        ),
        mesh=mesh,
        compiler_params=pltpu.CompilerParams(
            use_tc_tiling_on_sc=False, needs_layout_passes=False),
        scratch_types=[
            pltpu.VMEM((tok_pw,), jnp.int32),
            pltpu.VMEM((tok_pw,), jnp.int32),
            pltpu.VMEM((tok_pw,), jnp.int32),
            pltpu.VMEM((2 * d * vrows,), jnp.float32),
            pltpu.VMEM((b,), jnp.int32),
            pltpu.VMEM((hd, b), jnp.float32),
            pltpu.VMEM((hd, b), jnp.float32),
            [pltpu.VMEM((_CH, d), jnp.float32)] * 2,
            [pltpu.SemaphoreType.DMA] * 2,
            [pltpu.SemaphoreType.DMA] * 2,
        ],
    )
    def k(x_hbm, m_hbm, d_hbm, tm_hbm, tok_hbm, maug_hbm, daug_hbm, kvt_hbm,
          h_out, k_out, v_out,
          xi_v, mi_v, di_v, kvt_v, idx_v, kst, vst, hbufs, gsems, wsems):
        wid = lax.axis_index("s") * nc + lax.axis_index("c")
        tbase = wid * tok_pw
        pltpu.sync_copy(x_hbm.at[pl.ds(tbase, tok_pw)], xi_v)
        pltpu.sync_copy(m_hbm.at[pl.ds(tbase, tok_pw)], mi_v)
        pltpu.sync_copy(d_hbm.at[pl.ds(tbase, tok_pw)], di_v)

        def wait_write(buf, out_hbm, sem):
            pltpu.make_async_copy(buf, out_hbm.at[pl.ds(0, buf.shape[0])], sem).wait()

        for c in range(n_tok_ch):
            i = c % 2
            off = c * _CH
            if c >= 2:
                pltpu.make_async_copy(hbufs[i], h_out.at[pl.ds(0, _CH)], wsems[i]).wait()
            pltpu.async_copy(tok_hbm.at[xi_v.at[pl.ds(off, _CH)]], hbufs[i], gsems[i]).wait()
            a = pltpu.async_copy(maug_hbm.at[mi_v.at[pl.ds(off, _CH)]], hbufs[i], gsems[i], add=True)
            bb = pltpu.async_copy(daug_hbm.at[di_v.at[pl.ds(off, _CH)]], hbufs[i], gsems[i], add=True)
            a.wait()
            bb.wait()
            pltpu.async_copy(hbufs[i], h_out.at[pl.ds(tbase + off, _CH)], wsems[i])
        pltpu.sync_copy(kvt_hbm, kvt_v)
        for c in range(max(0, n_tok_ch - 2), n_tok_ch):
            pltpu.make_async_copy(hbufs[c % 2], h_out.at[pl.ds(0, _CH)], wsems[c % 2]).wait()

        def fill(st, base, col, idx16):
            for dd in range(hd):
                val = plsc.load_gather(kvt_v, [idx16 + (base + dd * vrows)])
                st[dd, pl.ds(col, 16)] = val

        def unit_body(t, carry):
            u = wid * n_units + t
            slab = u // 2
            d0 = (u % 2) * hd
            pltpu.sync_copy(tm_hbm.at[slab], idx_v)
            kbase = d0 * vrows
            vbase = (d + d0) * vrows

            @pl.when(t > 0)
            def _():
                wait_write(kst, k_out.at[0], wsems[0])

            def bg_k(bg, carry2):
                col = bg * 16
                fill(kst, kbase, col, idx_v[pl.ds(col, 16)])
                return carry2

            lax.fori_loop(0, nbg, bg_k, 0)
            pltpu.async_copy(kst, k_out.at[slab, pl.ds(d0, hd)], wsems[0])

            @pl.when(t > 0)
            def _():
                wait_write(vst, v_out.at[0], wsems[1])

            def bg_v(bg, carry2):
                col = bg * 16
                fill(vst, vbase, col, idx_v[pl.ds(col, 16)])
                return carry2

            lax.fori_loop(0, nbg, bg_v, 0)
            pltpu.async_copy(vst, v_out.at[slab, pl.ds(d0, hd)], wsems[1])
            return carry

        lax.fori_loop(0, n_units, unit_body, 0)
        wait_write(kst, k_out.at[0], wsems[0])
        wait_write(vst, v_out.at[0], wsems[1])

    return k(x_flat, midx, didx, tm_lb, tok_table, maug, daug, kvt_flat)


def _layernorm_tc(h, gamma, beta):
    n, d = h.shape
    blk = 1024

    def body(h_ref, g_ref, b_ref, o_ref):
        hv = h_ref[...]
        u = jnp.mean(hv, axis=-1, keepdims=True)
        c = hv - u
        s = jnp.mean(c * c, axis=-1, keepdims=True)
        o_ref[...] = g_ref[...] * (c * lax.rsqrt(s + _EPS)) + b_ref[...]

    return pl.pallas_call(
        body,
        grid=(n // blk,),
        in_specs=[
            pl.BlockSpec((blk, d), lambda i: (i, 0)),
            pl.BlockSpec((1, d), lambda i: (0, 0)),
            pl.BlockSpec((1, d), lambda i: (0, 0)),
        ],
        out_specs=pl.BlockSpec((blk, d), lambda i: (i, 0)),
        out_shape=jax.ShapeDtypeStruct((n, d), jnp.float32),
    )(h, gamma.reshape(1, d), beta.reshape(1, d))


def kernel(x, stamp, time_matrix, tok_table, weekday_table, day_table, month_table, K_table, V_table, gamma, beta):
    b, l = x.shape
    d = tok_table.shape[1]

    x_flat = x.reshape(-1)
    tm_lb = jnp.transpose(time_matrix, (1, 2, 0)).reshape(l * l, b)
    kvt = jnp.concatenate([K_table.T, V_table.T], axis=0)
    m_sent = month_table.shape[0]
    d_sent = day_table.shape[0]
    midx = jnp.concatenate(
        [jnp.full((b, 1), m_sent, jnp.int32), stamp[:, :, 0]], axis=1).reshape(-1)
    didx = jnp.concatenate(
        [jnp.full((b, 1), d_sent, jnp.int32), stamp[:, :, 1]], axis=1).reshape(-1)
    maug = jnp.concatenate(
        [month_table, jnp.zeros((3, d), month_table.dtype)], axis=0)
    daug = jnp.concatenate(
        [day_table, jnp.zeros((8, d), day_table.dtype)], axis=0)

    h, kout, vout = _sc_gather(x_flat, midx, didx, tm_lb, tok_table,
                               maug, daug, kvt)
    out = _layernorm_tc(h, gamma, beta)
    kf = jnp.transpose(kout.reshape(l, l, d, b), (3, 0, 1, 2))
    vf = jnp.transpose(vout.reshape(l, l, d, b), (3, 0, 1, 2))
    return (out.reshape(b, l, d), kf, vf)
